# baseline (speedup 1.0000x reference)
import jax
import jax.numpy as jnp
from jax.experimental import pallas as pl
from jax.experimental.pallas import tpu as pltpu
from jax.experimental.pallas import tpu_sc as plsc

def _dot_like_ref(a, b, dim_numbers=None):
  a16 = a.astype(jnp.bfloat16)
  b16 = b.astype(jnp.bfloat16)
  if dim_numbers is None:
    dim_numbers = (((a.ndim - 1,), (0,)), ((), ()))
  return jax.lax.dot_general(a16, b16, dim_numbers,
                             preferred_element_type=jnp.float32)


def _elu(u):
  return jnp.where(u > 0, u, jnp.exp(u) - 1.0)


def _rowsum32_like_ref(v):
  a = ((v[:, 0:8] + v[:, 8:16]) + v[:, 16:24]) + v[:, 24:32]
  a = a[:, 0:4] + a[:, 4:8]
  a = a[:, 0:2] + a[:, 2:4]
  return a[:, 0:1] + a[:, 1:2]


def _vq_body(z_ref, cb_ref, idx_ref, c2_ref):
  @pl.when(pl.program_id(0) == 0)
  def _():
    cb0 = cb_ref[...]
    c2_ref[...] = jnp.sum(cb0 * cb0, axis=1)[None, :]

  z = z_ref[...]
  t = _dot_like_ref(z, cb_ref[...], (((1,), (1,)), ((), ())))
  z2 = _rowsum32_like_ref(z * z)
  d2 = z2 + c2_ref[...] - 2.0 * t
  d = jnp.sqrt(jnp.maximum(d2, 0.0))
  dmin = jnp.min(d, axis=1, keepdims=True)
  k = d.shape[1]
  iota = jax.lax.broadcasted_iota(jnp.int32, d.shape, 1)
  idx = jnp.min(jnp.where(d == dmin, iota, k), axis=1)
  idx_ref[0, 0, :] = idx


def _vq_argmin(z, cb, row_block=256):
  n, e = z.shape
  k, _ = cb.shape
  nb = n // row_block
  idx3 = pl.pallas_call(
      _vq_body,
      grid=(nb,),
      in_specs=[
          pl.BlockSpec((row_block, e), lambda i: (i, 0)),
          pl.BlockSpec((k, e), lambda i: (0, 0)),
      ],
      out_specs=pl.BlockSpec((1, 1, row_block), lambda i: (i, 0, 0)),
      out_shape=jax.ShapeDtypeStruct((nb, 1, row_block), jnp.int32),
      scratch_shapes=[pltpu.VMEM((1, k), jnp.float32)],
  )(z, cb)
  return idx3.reshape(n)


_SPLIT = 384


def _dec_body(cb_ref, w3_ref, b3_ref, w4_ref, b4_ref, out_a_ref, out_b_ref):
  h2 = _elu(_dot_like_ref(cb_ref[...], w3_ref[...]) + b3_ref[...])
  out = jax.nn.sigmoid(_dot_like_ref(h2, w4_ref[...]) + b4_ref[...])
  out_a_ref[...] = out[:, :_SPLIT]
  out_b_ref[...] = out[:, _SPLIT:]


def _decode_table(cb, w3, b3, w4, b4, row_block=1024):
  k, e = cb.shape
  hdim = w3.shape[1]
  f = w4.shape[1]
  nb = k // row_block
  return pl.pallas_call(
      _dec_body,
      grid=(nb,),
      in_specs=[
          pl.BlockSpec((row_block, e), lambda i: (i, 0)),
          pl.BlockSpec((e, hdim), lambda i: (0, 0)),
          pl.BlockSpec((1, hdim), lambda i: (0, 0)),
          pl.BlockSpec((hdim, f), lambda i: (0, 0)),
          pl.BlockSpec((1, f), lambda i: (0, 0)),
      ],
      out_specs=[pl.BlockSpec((row_block, _SPLIT), lambda i: (i, 0)),
                 pl.BlockSpec((row_block, f - _SPLIT), lambda i: (i, 0))],
      out_shape=[jax.ShapeDtypeStruct((k, _SPLIT), jnp.float32),
                 jax.ShapeDtypeStruct((k, f - _SPLIT), jnp.float32)],
  )(cb, w3, b3.reshape(1, -1), w4, b4.reshape(1, -1))


_SC_MESH = dict(core_axis_name="c", subcore_axis_name="s")


def _gather1_sc(table, idx, window=128):
  n = idx.shape[1]
  w = table.shape[1]
  mesh = plsc.VectorSubcoreMesh(**_SC_MESH)

  @pl.kernel(out_type=jax.ShapeDtypeStruct((n, w), table.dtype), mesh=mesh)
  def gather_kernel(t_hbm, i_hbm, o_hbm):
    def body(i_vmem, o_vmem):
      pltpu.sync_copy(t_hbm.at[i_vmem.at[0]], o_vmem)

    pltpu.emit_pipeline(
        body,
        grid=(n // window,),
        in_specs=[pl.BlockSpec((1, window), lambda i: (0, i))],
        out_specs=[pl.BlockSpec((window, w), lambda i: (i, 0))],
        core_axis_name=("c", "s"),
        dimension_semantics=(pltpu.PARALLEL,),
    )(i_hbm, o_hbm)

  return gather_kernel(table, idx)


def _gather2_sc(table_a, table_b, idx, window=128):
  n = idx.shape[1]
  wa = table_a.shape[1]
  wb = table_b.shape[1]
  mesh = plsc.VectorSubcoreMesh(**_SC_MESH)

  @pl.kernel(out_type=(jax.ShapeDtypeStruct((n, wa), table_a.dtype),
                       jax.ShapeDtypeStruct((n, wb), table_b.dtype)),
             mesh=mesh)
  def gather_kernel(a_hbm, b_hbm, i_hbm, oa_hbm, ob_hbm):
    def body(i_vmem, oa_vmem, ob_vmem):
      pltpu.sync_copy(a_hbm.at[i_vmem.at[0]], oa_vmem)
      pltpu.sync_copy(b_hbm.at[i_vmem.at[0]], ob_vmem)

    pltpu.emit_pipeline(
        body,
        grid=(n // window,),
        in_specs=[pl.BlockSpec((1, window), lambda i: (0, i))],
        out_specs=[pl.BlockSpec((window, wa), lambda i: (i, 0)),
                   pl.BlockSpec((window, wb), lambda i: (i, 0))],
        core_axis_name=("c", "s"),
        dimension_semantics=(pltpu.PARALLEL,),
    )(i_hbm, oa_hbm, ob_hbm)

  return gather_kernel(table_a, table_b, idx)


def kernel(x, W1, b1, W2, b2, codebook, W3, b3, W4, b4):
  f = W4.shape[1]
  e = codebook.shape[1]
  fpad = -f % 128
  epad = -e % 128
  z = jax.nn.elu(x @ W1 + b1) @ W2 + b2
  idx = _vq_argmin(z, codebook)
  i2 = idx.reshape(1, -1)
  w4p = jnp.pad(W4, ((0, 0), (0, fpad)))
  b4p = jnp.pad(b4, (0, fpad))
  dec_a, dec_b = _decode_table(codebook, W3, b3, w4p, b4p)
  cbp = jnp.pad(codebook, ((0, 0), (0, epad)))
  xr_a = _gather1_sc(dec_a, i2)
  xr_b, zq_pad = _gather2_sc(dec_b, cbp, i2)
  x_recon = jnp.concatenate([xr_a, xr_b], axis=1)[:, :f]
  return x_recon, z, zq_pad[:, :e]


# speedup vs baseline: 1.2771x; 1.2771x over previous
import functools

import jax
import jax.numpy as jnp
from jax.experimental import pallas as pl
from jax.experimental.pallas import tpu as pltpu
from jax.experimental.pallas import tpu_sc as plsc

def _dot_like_ref(a, b, dim_numbers=None):
  a16 = a.astype(jnp.bfloat16)
  b16 = b.astype(jnp.bfloat16)
  if dim_numbers is None:
    dim_numbers = (((a.ndim - 1,), (0,)), ((), ()))
  return jax.lax.dot_general(a16, b16, dim_numbers,
                             preferred_element_type=jnp.float32)


def _elu(u):
  return jnp.where(u > 0, u, jnp.exp(u) - 1.0)


def _rowsum32_like_ref(v):
  a = ((v[:, 0:8] + v[:, 8:16]) + v[:, 16:24]) + v[:, 24:32]
  a = a[:, 0:4] + a[:, 4:8]
  a = a[:, 0:2] + a[:, 2:4]
  return a[:, 0:1] + a[:, 1:2]


def _vq_body(z_ref, cb_ref, idx_ref, c2_ref):
  @pl.when(pl.program_id(0) == 0)
  def _():
    cb0 = cb_ref[...]
    c2_ref[...] = jnp.sum(cb0 * cb0, axis=1)[None, :]

  z = z_ref[...]
  t = _dot_like_ref(z, cb_ref[...], (((1,), (1,)), ((), ())))
  z2 = _rowsum32_like_ref(z * z)
  d2 = z2 + c2_ref[...] - 2.0 * t
  d = jnp.sqrt(jnp.maximum(d2, 0.0))
  dmin = jnp.min(d, axis=1, keepdims=True)
  k = d.shape[1]
  iota = jax.lax.broadcasted_iota(jnp.int32, d.shape, 1)
  idx = jnp.min(jnp.where(d == dmin, iota, k), axis=1)
  idx_ref[0, 0, :] = idx


def _vq_argmin(z, cb, row_block=256):
  n, e = z.shape
  k, _ = cb.shape
  nb = n // row_block
  idx3 = pl.pallas_call(
      _vq_body,
      grid=(nb,),
      in_specs=[
          pl.BlockSpec((row_block, e), lambda i: (i, 0)),
          pl.BlockSpec((k, e), lambda i: (0, 0)),
      ],
      out_specs=pl.BlockSpec((1, 1, row_block), lambda i: (i, 0, 0)),
      out_shape=jax.ShapeDtypeStruct((nb, 1, row_block), jnp.int32),
      scratch_shapes=[pltpu.VMEM((1, k), jnp.float32)],
  )(z, cb)
  return idx3.reshape(n)


def _dec_body(zq_ref, w3_ref, b3_ref, w4_ref, b4_ref, out_ref, e):
  zq = zq_ref[...][:, :e]
  h2 = _elu(_dot_like_ref(zq, w3_ref[...]) + b3_ref[...])
  out_ref[...] = jax.nn.sigmoid(_dot_like_ref(h2, w4_ref[...]) + b4_ref[...])


def _decode(zq_pad, w3, b3, w4, b4, row_block=512):
  n = zq_pad.shape[0]
  e, hdim = w3.shape
  f = w4.shape[1]
  nb = n // row_block
  ep = zq_pad.shape[1]
  return pl.pallas_call(
      functools.partial(_dec_body, e=e),
      grid=(nb,),
      in_specs=[
          pl.BlockSpec((row_block, ep), lambda i: (i, 0)),
          pl.BlockSpec((e, hdim), lambda i: (0, 0)),
          pl.BlockSpec((1, hdim), lambda i: (0, 0)),
          pl.BlockSpec((hdim, f), lambda i: (0, 0)),
          pl.BlockSpec((1, f), lambda i: (0, 0)),
      ],
      out_specs=pl.BlockSpec((row_block, f), lambda i: (i, 0)),
      out_shape=jax.ShapeDtypeStruct((n, f), jnp.float32),
  )(zq_pad, w3, b3.reshape(1, -1), w4, b4.reshape(1, -1))


_SC_MESH = dict(core_axis_name="c", subcore_axis_name="s")


def _gather1_sc(table, idx, window=128):
  n = idx.shape[1]
  w = table.shape[1]
  mesh = plsc.VectorSubcoreMesh(**_SC_MESH)

  @pl.kernel(out_type=jax.ShapeDtypeStruct((n, w), table.dtype), mesh=mesh)
  def gather_kernel(t_hbm, i_hbm, o_hbm):
    def body(i_vmem, o_vmem):
      pltpu.sync_copy(t_hbm.at[i_vmem.at[0]], o_vmem)

    pltpu.emit_pipeline(
        body,
        grid=(n // window,),
        in_specs=[pl.BlockSpec((1, window), lambda i: (0, i))],
        out_specs=[pl.BlockSpec((window, w), lambda i: (i, 0))],
        core_axis_name=("c", "s"),
        dimension_semantics=(pltpu.PARALLEL,),
    )(i_hbm, o_hbm)

  return gather_kernel(table, idx)


def kernel(x, W1, b1, W2, b2, codebook, W3, b3, W4, b4):
  e = codebook.shape[1]
  epad = -e % 128
  z = jax.nn.elu(x @ W1 + b1) @ W2 + b2
  idx = _vq_argmin(z, codebook)
  i2 = idx.reshape(1, -1)
  cbp = jnp.pad(codebook, ((0, 0), (0, epad)))
  zq_pad = _gather1_sc(cbp, i2)
  x_recon = _decode(zq_pad, W3, b3, W4, b4)
  return x_recon, z, zq_pad[:, :e]


# speedup vs baseline: 1.6280x; 1.2748x over previous
import functools

import jax
import jax.numpy as jnp
from jax.experimental import pallas as pl
from jax.experimental.pallas import tpu as pltpu
from jax.experimental.pallas import tpu_sc as plsc

def _dot_like_ref(a, b, dim_numbers=None):
  a16 = a.astype(jnp.bfloat16)
  b16 = b.astype(jnp.bfloat16)
  if dim_numbers is None:
    dim_numbers = (((a.ndim - 1,), (0,)), ((), ()))
  return jax.lax.dot_general(a16, b16, dim_numbers,
                             preferred_element_type=jnp.float32)


def _elu(u):
  return jnp.where(u > 0, u, jnp.exp(u) - 1.0)


def _rowsum32_like_ref(v):
  a = ((v[:, 0:8] + v[:, 8:16]) + v[:, 16:24]) + v[:, 24:32]
  a = a[:, 0:4] + a[:, 4:8]
  a = a[:, 0:2] + a[:, 2:4]
  return a[:, 0:1] + a[:, 1:2]


def _vq_body(z_ref, cb_ref, idx_ref, c2_ref):
  @pl.when(pl.program_id(0) == 0)
  def _():
    cb0 = cb_ref[...]
    c2_ref[...] = jnp.sum(cb0 * cb0, axis=1)[None, :]

  z = z_ref[...]
  t = _dot_like_ref(z, cb_ref[...], (((1,), (1,)), ((), ())))
  z2 = _rowsum32_like_ref(z * z)
  d2 = z2 + c2_ref[...] - 2.0 * t
  m2 = jnp.min(d2, axis=1, keepdims=True)
  s = jnp.sqrt(jnp.maximum(m2, 0.0))
  c = s * s
  ci = jax.lax.bitcast_convert_type(c, jnp.int32)
  hi = m2
  for step in range(-4, 6):
    cand = jax.lax.bitcast_convert_type(ci + step, jnp.float32)
    ok = jnp.sqrt(jnp.maximum(cand, 0.0)) == s
    hi = jnp.where(ok & (cand > hi), cand, hi)
  hi = jnp.where(s == 0.0, 0.0, hi)
  k = d2.shape[1]
  iota = jax.lax.broadcasted_iota(jnp.int32, d2.shape, 1)
  idx = jnp.min(jnp.where(d2 <= hi, iota, k), axis=1)
  idx_ref[0, 0, :] = idx


def _vq_argmin(z, cb, row_block=256):
  n, e = z.shape
  k, _ = cb.shape
  nb = n // row_block
  idx3 = pl.pallas_call(
      _vq_body,
      grid=(nb,),
      in_specs=[
          pl.BlockSpec((row_block, e), lambda i: (i, 0)),
          pl.BlockSpec((k, e), lambda i: (0, 0)),
      ],
      out_specs=pl.BlockSpec((1, 1, row_block), lambda i: (i, 0, 0)),
      out_shape=jax.ShapeDtypeStruct((nb, 1, row_block), jnp.int32),
      scratch_shapes=[pltpu.VMEM((1, k), jnp.float32)],
  )(z, cb)
  return idx3.reshape(n)


def _dec_body(zq_ref, w3_ref, b3_ref, w4_ref, b4_ref, out_ref, e):
  zq = zq_ref[...][:, :e]
  h2 = _elu(_dot_like_ref(zq, w3_ref[...]) + b3_ref[...])
  out_ref[...] = jax.nn.sigmoid(_dot_like_ref(h2, w4_ref[...]) + b4_ref[...])


def _decode(zq_pad, w3, b3, w4, b4, row_block=512):
  n = zq_pad.shape[0]
  e, hdim = w3.shape
  f = w4.shape[1]
  nb = n // row_block
  ep = zq_pad.shape[1]
  return pl.pallas_call(
      functools.partial(_dec_body, e=e),
      grid=(nb,),
      in_specs=[
          pl.BlockSpec((row_block, ep), lambda i: (i, 0)),
          pl.BlockSpec((e, hdim), lambda i: (0, 0)),
          pl.BlockSpec((1, hdim), lambda i: (0, 0)),
          pl.BlockSpec((hdim, f), lambda i: (0, 0)),
          pl.BlockSpec((1, f), lambda i: (0, 0)),
      ],
      out_specs=pl.BlockSpec((row_block, f), lambda i: (i, 0)),
      out_shape=jax.ShapeDtypeStruct((n, f), jnp.float32),
  )(zq_pad, w3, b3.reshape(1, -1), w4, b4.reshape(1, -1))


_SC_MESH = dict(core_axis_name="c", subcore_axis_name="s")


def _gather1_sc(table, idx, window=128):
  n = idx.shape[1]
  w = table.shape[1]
  mesh = plsc.VectorSubcoreMesh(**_SC_MESH)

  @pl.kernel(out_type=jax.ShapeDtypeStruct((n, w), table.dtype), mesh=mesh)
  def gather_kernel(t_hbm, i_hbm, o_hbm):
    def body(i_vmem, o_vmem):
      pltpu.sync_copy(t_hbm.at[i_vmem.at[0]], o_vmem)

    pltpu.emit_pipeline(
        body,
        grid=(n // window,),
        in_specs=[pl.BlockSpec((1, window), lambda i: (0, i))],
        out_specs=[pl.BlockSpec((window, w), lambda i: (i, 0))],
        core_axis_name=("c", "s"),
        dimension_semantics=(pltpu.PARALLEL,),
    )(i_hbm, o_hbm)

  return gather_kernel(table, idx)


def kernel(x, W1, b1, W2, b2, codebook, W3, b3, W4, b4):
  e = codebook.shape[1]
  epad = -e % 128
  z = jax.nn.elu(x @ W1 + b1) @ W2 + b2
  idx = _vq_argmin(z, codebook)
  i2 = idx.reshape(1, -1)
  cbp = jnp.pad(codebook, ((0, 0), (0, epad)))
  zq_pad = _gather1_sc(cbp, i2)
  x_recon = _decode(zq_pad, W3, b3, W4, b4)
  return x_recon, z, zq_pad[:, :e]
